# Initial kernel scaffold; baseline (speedup 1.0000x reference)
import functools

import jax
import jax.numpy as jnp
from jax import lax
from jax.experimental import pallas as pl
from jax.experimental.pallas import tpu as pltpu
from jax.experimental.pallas import tpu_sc as plsc

_N = 10000
_E = 320000
_D = 128
_T = 16
_EA = 16
_K = 5
_NPAD = 10240
_NC, _NS = 2, 16
_CG = 80
_EWG = 2 * _E // (_NC * _NS)
_NCHG = _EWG // _CG
_ET = _E // _NS
_NCH = _ET // _CG
_NSL = _NPAD // _NS


def _node_mm_body(x_ref, wa_ref, wb_ref, a_ref, b_ref):
    xb = x_ref[...]
    a_ref[...] = jnp.dot(xb, wa_ref[...], preferred_element_type=jnp.float32)
    b_ref[...] = jnp.dot(xb, wb_ref[...], preferred_element_type=jnp.float32)


def _node_transform(x, wa_t, wb_t):
    bn = 1000
    return pl.pallas_call(
        _node_mm_body,
        grid=(_N // bn,),
        in_specs=[
            pl.BlockSpec((bn, _D), lambda i: (i, 0)),
            pl.BlockSpec((_D, _D), lambda i: (0, 0)),
            pl.BlockSpec((_D, _D), lambda i: (0, 0)),
        ],
        out_specs=[pl.BlockSpec((bn, _D), lambda i: (i, 0))] * 2,
        out_shape=[jax.ShapeDtypeStruct((_N, _D), jnp.float32)] * 2,
    )(x, wa_t, wb_t)


def _gather_body(src_hbm, dst_hbm, a_hbm, b_hbm, gs_hbm, gd_hbm,
                 srcv, dstv, rs, rd, sem1, sem2):
    wid = lax.axis_index("s") * _NC + lax.axis_index("c")
    base = wid * _EWG
    pltpu.sync_copy(src_hbm.at[wid], srcv)
    pltpu.sync_copy(dst_hbm.at[wid], dstv)

    def body(k, carry):
        c1 = pltpu.async_copy(a_hbm.at[srcv.at[k]], rs, sem1)
        c2 = pltpu.async_copy(b_hbm.at[dstv.at[k]], rd, sem2)
        c1.wait()
        c2.wait()
        off = base + k * _CG
        pltpu.sync_copy(rs, gs_hbm.at[pl.ds(off, _CG)])
        pltpu.sync_copy(rd, gd_hbm.at[pl.ds(off, _CG)])
        return carry

    lax.fori_loop(0, _NCHG, body, 0)


def _gather_rows(src32, dst32, a_nd, b_nd):
    mesh = plsc.VectorSubcoreMesh(
        core_axis_name="c", subcore_axis_name="s",
        num_cores=_NC, num_subcores=_NS)
    f = pl.kernel(
        _gather_body,
        out_type=[jax.ShapeDtypeStruct((2 * _E, _D), jnp.float32)] * 2,
        mesh=mesh,
        scratch_types=[
            pltpu.VMEM((_NCHG, _CG), jnp.int32),
            pltpu.VMEM((_NCHG, _CG), jnp.int32),
            pltpu.VMEM((_CG, _D), jnp.float32),
            pltpu.VMEM((_CG, _D), jnp.float32),
            pltpu.SemaphoreType.DMA,
            pltpu.SemaphoreType.DMA,
        ],
    )
    return f(src32, dst32, a_nd, b_nd)


def _edge_mlp_body(gs0, gd0, gs1, gd1, at0, at1, b1r, w2t, b2r,
                   wh0, wh1, wl0, wl1, be0, be1,
                   s0, s1, a0, a1):
    w2 = w2t[...]
    b1v = b1r[...]
    b2v = b2r[...]
    h0 = jnp.maximum(gs0[...] + gd0[...] + b1v, 0.0)
    z0 = jnp.dot(h0, w2, preferred_element_type=jnp.float32) + b2v
    s0[...] = jnp.dot(jax.nn.sigmoid(z0), wh0[...],
                      preferred_element_type=jnp.float32)
    h1 = jnp.maximum(gs1[...] + gd1[...] + b1v, 0.0)
    z1 = jnp.dot(h1, w2, preferred_element_type=jnp.float32) + b2v
    s1[...] = jnp.dot(jax.nn.sigmoid(z1), wh1[...],
                      preferred_element_type=jnp.float32)
    a0[...] = jnp.dot(at0[...], wl0[...],
                      preferred_element_type=jnp.float32) + be0[...]
    a1[...] = jnp.dot(at1[...], wl1[...],
                      preferred_element_type=jnp.float32) + be1[...]


def _edge_mlp(gs, gd, attr0, attr1, b1r, w2t, b2r, wh0, wh1, wl0, wl1,
              be0, be1):
    be = 1000
    nb = _E // be
    rep = lambda shape: pl.BlockSpec(shape, lambda i: (0, 0))
    return pl.pallas_call(
        _edge_mlp_body,
        grid=(nb,),
        in_specs=[
            pl.BlockSpec((be, _D), lambda i: (i, 0)),
            pl.BlockSpec((be, _D), lambda i: (i, 0)),
            pl.BlockSpec((be, _D), lambda i: (i + _E // 1000, 0)),
            pl.BlockSpec((be, _D), lambda i: (i + _E // 1000, 0)),
            pl.BlockSpec((be, _EA), lambda i: (i, 0)),
            pl.BlockSpec((be, _EA), lambda i: (i, 0)),
            rep((1, _D)), rep((_D, _T)), rep((1, _T)),
            rep((_T, 1)), rep((_T, 1)), rep((_EA, 1)), rep((_EA, 1)),
            rep((1, 1)), rep((1, 1)),
        ],
        out_specs=[pl.BlockSpec((be, 1), lambda i: (i, 0))] * 4,
        out_shape=[jax.ShapeDtypeStruct((_E, 1), jnp.float32)] * 4,
    )(gs, gd, gs, gd, attr0, attr1, b1r, w2t, b2r, wh0, wh1, wl0, wl1,
      be0, be1)


def _appnp_body(sflat, br4, a4, row4, col4, maskp, alpha2, oma2,
                fill,
                x1, x2, x3, x4, outv, tmpv, swv, sba, sbb, zbuf, mbuf,
                alv, omv, sem, agg, out_s):
    c = lax.axis_index("c")
    t = lax.axis_index("s")
    pltpu.sync_copy(alpha2.at[c], alv)
    pltpu.sync_copy(oma2.at[c], omv)
    a16 = alv[...]
    om16 = omv[...]
    zero16 = jnp.zeros((16,), jnp.float32)

    def zl(v, u):
        zbuf[pl.ds(v * 16, 16)] = zero16
        return u
    lax.fori_loop(0, _NSL // 16, zl, 0)

    pltpu.sync_copy(br4.at[c, t], x1)
    pltpu.sync_copy(a4.at[c, t], x3)

    def gath(k, u):
        pltpu.async_copy(sflat.at[x1.at[k]],
                         x4.at[pl.ds(k * _CG, _CG)], sem).wait()
        return u
    lax.fori_loop(0, _NCH, gath, 0)

    def ewl(k, u):
        for v in range(_CG // 16):
            sl = pl.ds(v * 16, 16)
            s16 = x4[pl.ds(k * _CG + v * 16, 16)]
            val = x3[k, sl] + s16
            x3[k, sl] = 1.0 / (1.0 + jnp.exp(-val))
        return u
    lax.fori_loop(0, _NCH, ewl, 0)

    pltpu.sync_copy(zbuf, agg.at[pl.ds(t * _NSL, _NSL)])
    plsc.subcore_barrier()
    pltpu.sync_copy(col4.at[c, t], x2)

    def scat_deg(k, u):
        pltpu.sync_copy(x3.at[k], agg.at[x2.at[k]], add=True)
        return u
    lax.fori_loop(0, _NCH, scat_deg, 0)
    plsc.subcore_barrier()

    pltpu.sync_copy(agg, x4.at[pl.ds(0, _NPAD)])

    def dl(v, u):
        sl = pl.ds(v * 16, 16)
        d = x4[sl] + 1.0
        i = jnp.int32(0x5F3759DF) - lax.shift_right_logical(
            plsc.bitcast(d, jnp.int32), 1)
        y = plsc.bitcast(i, jnp.float32)
        y = y * (1.5 - 0.5 * d * y * y)
        y = y * (1.5 - 0.5 * d * y * y)
        y = y * (1.5 - 0.5 * d * y * y)
        tmpv[sl] = y
        swv[sl] = om16 * y * y
        return u
    lax.fori_loop(0, _NPAD // 16, dl, 0)
    plsc.subcore_barrier()

    pltpu.sync_copy(row4.at[c, t], x1)

    def nl(k, u):
        for v in range(_CG // 16):
            sl = pl.ds(v * 16, 16)
            r = x1[k, sl]
            cc = x2[k, sl]
            dr = plsc.load_gather(tmpv, [r])
            dc = plsc.load_gather(tmpv, [cc])
            x3[k, sl] = om16 * dr * x3[k, sl] * dc
        return u
    lax.fori_loop(0, _NCH, nl, 0)

    pltpu.sync_copy(maskp, outv)

    def hl(v, u):
        sl = pl.ds(v * 16, 16)
        r = jnp.maximum(outv[sl], 0.0)
        outv[sl] = r
        tmpv[sl] = a16 * r
        return u
    lax.fori_loop(0, _NPAD // 16, hl, 0)

    for _hop in range(_K):
        pltpu.sync_copy(zbuf, agg.at[pl.ds(t * _NSL, _NSL)])
        plsc.subcore_barrier()

        def scat(k, u):
            for v in range(_CG // 16):
                sl = pl.ds(v * 16, 16)
                r = x1[k, sl]
                mbuf[sl] = x3[k, sl] * plsc.load_gather(outv, [r])
            pltpu.sync_copy(mbuf, agg.at[x2.at[k]], add=True)
            return u
        lax.fori_loop(0, _NCH, scat, 0)
        plsc.subcore_barrier()

        base = t * _NSL
        pltpu.sync_copy(agg.at[pl.ds(base, _NSL)], sba)

        def upd(v, u):
            sl = pl.ds(v * 16, 16)
            gl = pl.ds(base + v * 16, 16)
            sbb[sl] = sba[sl] + swv[gl] * outv[gl] + tmpv[gl]
            return u
        lax.fori_loop(0, _NSL // 16, upd, 0)
        pltpu.sync_copy(sbb, out_s.at[pl.ds(base, _NSL)])
        plsc.subcore_barrier()
        pltpu.sync_copy(out_s, outv)

    pltpu.sync_copy(outv.at[pl.ds(t * _NSL, _NSL)],
                    fill.at[c, pl.ds(t * _NSL, _NSL)])


def _appnp(sflat, br4, a4, row4, col4, maskp, alpha2, oma2):
    mesh = plsc.VectorSubcoreMesh(
        core_axis_name="c", subcore_axis_name="s",
        num_cores=_NC, num_subcores=_NS)
    f = pl.kernel(
        _appnp_body,
        out_type=jax.ShapeDtypeStruct((_NC, _NPAD), jnp.float32),
        mesh=mesh,
        scratch_types=[
            pltpu.VMEM((_NCH, _CG), jnp.int32),
            pltpu.VMEM((_NCH, _CG), jnp.int32),
            pltpu.VMEM((_NCH, _CG), jnp.float32),
            pltpu.VMEM((_ET,), jnp.float32),
            pltpu.VMEM((_NPAD,), jnp.float32),
            pltpu.VMEM((_NPAD,), jnp.float32),
            pltpu.VMEM((_NPAD,), jnp.float32),
            pltpu.VMEM((_NSL,), jnp.float32),
            pltpu.VMEM((_NSL,), jnp.float32),
            pltpu.VMEM((_NSL,), jnp.float32),
            pltpu.VMEM((_CG,), jnp.float32),
            pltpu.VMEM((16,), jnp.float32),
            pltpu.VMEM((16,), jnp.float32),
            pltpu.SemaphoreType.DMA,
            pltpu.VMEM_SHARED((_NPAD,), jnp.float32),
            pltpu.VMEM_SHARED((_NPAD,), jnp.float32),
        ],
    )
    return f(sflat, br4, a4, row4, col4, maskp, alpha2, oma2)


def kernel(x, mask, dom_spatial_edge_index, dom_dom_edge_index,
           proj_spatial_edge_index, proj_dom_edge_index,
           proj_br_spatial_edge_index, proj_br_dom_edge_index,
           proj_spatial_edge_attr, proj_dom_edge_attr,
           W1, b1, W2, b2,
           pos_We, pos_be, pos_alpha, pos_bias,
           dom_We, dom_be, dom_alpha, dom_bias):
    f32, i32 = jnp.float32, jnp.int32

    w1a = W1[:, :_D]
    w1b = W1[:, _D:2 * _D]
    w1c = W1[:, 2 * _D:]
    a_nd, b_nd = _node_transform(x, (w1a + w1c).T, (w1b - w1c).T)

    src = jnp.concatenate(
        [dom_spatial_edge_index[0], dom_dom_edge_index[0]]
    ).astype(i32).reshape(_NC * _NS, _NCHG, _CG)
    dst = jnp.concatenate(
        [dom_spatial_edge_index[1], dom_dom_edge_index[1]]
    ).astype(i32).reshape(_NC * _NS, _NCHG, _CG)
    gs, gd = _gather_rows(src, dst, a_nd, b_nd)

    s_sp, s_dom, a_sp, a_dom = _edge_mlp(
        gs, gd, proj_spatial_edge_attr, proj_dom_edge_attr,
        b1.reshape(1, _D), W2.T, b2.reshape(1, _T),
        pos_We[0, _EA:].reshape(_T, 1), dom_We[0, _EA:].reshape(_T, 1),
        pos_We[0, :_EA].reshape(_EA, 1), dom_We[0, :_EA].reshape(_EA, 1),
        pos_be.reshape(1, 1), dom_be.reshape(1, 1))

    sflat = jnp.concatenate([s_sp[:, 0], s_dom[:, 0]])
    br4 = jnp.stack([
        proj_br_spatial_edge_index.astype(i32),
        proj_br_dom_edge_index.astype(i32) + _E,
    ]).reshape(_NC, _NS, _NCH, _CG)
    a4 = jnp.stack([a_sp[:, 0], a_dom[:, 0]]).reshape(_NC, _NS, _NCH, _CG)
    row4 = jnp.stack([
        proj_spatial_edge_index[0], proj_dom_edge_index[0]
    ]).astype(i32).reshape(_NC, _NS, _NCH, _CG)
    col4 = jnp.stack([
        proj_spatial_edge_index[1], proj_dom_edge_index[1]
    ]).astype(i32).reshape(_NC, _NS, _NCH, _CG)
    maskp = jnp.pad(mask[:, 0].astype(f32), (0, _NPAD - _N))
    alph = jnp.stack([pos_alpha, dom_alpha]).astype(f32)[:, None]
    ones16 = jnp.ones((1, 16), f32)
    fill = _appnp(sflat, br4, a4, row4, col4, maskp,
                  alph * ones16, (1.0 - alph) * ones16)

    pos_fill = fill[0, :_N, None] - jax.nn.softplus(pos_bias)
    dom_fill = fill[1, :_N, None] - jax.nn.softplus(dom_bias)
    return jnp.maximum(mask, jnp.maximum(jnp.tanh(pos_fill),
                                         jnp.tanh(dom_fill)))


# speedup vs baseline: 13.3806x; 13.3806x over previous
import jax
import jax.numpy as jnp
from jax import lax
from jax.experimental import pallas as pl
from jax.experimental.pallas import tpu as pltpu
from jax.experimental.pallas import tpu_sc as plsc

_N = 10000
_E = 320000
_D = 128
_T = 16
_EA = 16
_K = 5
_NPAD = 10240
_NC, _NS = 2, 16
_CG = 128
_NW = _NC * _NS
_EWP = 20480
_NCHG = _EWP // _CG
_E2P = _NW * _EWP
_ETP = 20480
_NCH = _ETP // _CG
_EP = _NS * _ETP
_NSL = _NPAD // _NS


def _node_mm_body(x_ref, wa_ref, wb_ref, a_ref, b_ref):
    xb = x_ref[...]
    a_ref[...] = jnp.dot(xb, wa_ref[...], preferred_element_type=jnp.float32)
    b_ref[...] = jnp.dot(xb, wb_ref[...], preferred_element_type=jnp.float32)


def _node_transform(x, wa_t, wb_t):
    bn = 1000
    return pl.pallas_call(
        _node_mm_body,
        grid=(_N // bn,),
        in_specs=[
            pl.BlockSpec((bn, _D), lambda i: (i, 0)),
            pl.BlockSpec((_D, _D), lambda i: (0, 0)),
            pl.BlockSpec((_D, _D), lambda i: (0, 0)),
        ],
        out_specs=[pl.BlockSpec((bn, _D), lambda i: (i, 0))] * 2,
        out_shape=[jax.ShapeDtypeStruct((_N, _D), jnp.float32)] * 2,
    )(x, wa_t, wb_t)


def _gather_body(src_hbm, dst_hbm, a_hbm, b_hbm, gs_hbm, gd_hbm,
                 srcv, dstv, rs, rd, sem1, sem2):
    wid = lax.axis_index("s") * _NC + lax.axis_index("c")
    base = wid * _EWP
    pltpu.sync_copy(src_hbm.at[wid], srcv)
    pltpu.sync_copy(dst_hbm.at[wid], dstv)

    def body(k, carry):
        c1 = pltpu.async_copy(a_hbm.at[srcv.at[k]], rs, sem1)
        c2 = pltpu.async_copy(b_hbm.at[dstv.at[k]], rd, sem2)
        c1.wait()
        c2.wait()
        off = base + k * _CG
        pltpu.sync_copy(rs, gs_hbm.at[pl.ds(off, _CG)])
        pltpu.sync_copy(rd, gd_hbm.at[pl.ds(off, _CG)])
        return carry

    lax.fori_loop(0, _NCHG, body, 0)


def _gather_rows(src32, dst32, a_nd, b_nd):
    mesh = plsc.VectorSubcoreMesh(
        core_axis_name="c", subcore_axis_name="s",
        num_cores=_NC, num_subcores=_NS)
    f = pl.kernel(
        _gather_body,
        out_type=[jax.ShapeDtypeStruct((_E2P, _D), jnp.float32)] * 2,
        mesh=mesh,
        compiler_params=pltpu.CompilerParams(needs_layout_passes=False),
        scratch_types=[
            pltpu.VMEM((_NCHG, _CG), jnp.int32),
            pltpu.VMEM((_NCHG, _CG), jnp.int32),
            pltpu.VMEM((_CG, _D), jnp.float32),
            pltpu.VMEM((_CG, _D), jnp.float32),
            pltpu.SemaphoreType.DMA,
            pltpu.SemaphoreType.DMA,
        ],
    )
    return f(src32, dst32, a_nd, b_nd)


def _edge_mlp_body(gs0, gd0, gs1, gd1, at0, at1, b1r, w2t, b2r,
                   wh0, wh1, wl0, wl1, be0, be1,
                   s0, s1, a0, a1):
    w2 = w2t[...]
    b1v = b1r[...]
    b2v = b2r[...]
    h0 = jnp.maximum(gs0[...] + gd0[...] + b1v, 0.0)
    z0 = jnp.dot(h0, w2, preferred_element_type=jnp.float32) + b2v
    s0[...] = jnp.dot(jax.nn.sigmoid(z0), wh0[...],
                      preferred_element_type=jnp.float32)
    h1 = jnp.maximum(gs1[...] + gd1[...] + b1v, 0.0)
    z1 = jnp.dot(h1, w2, preferred_element_type=jnp.float32) + b2v
    s1[...] = jnp.dot(jax.nn.sigmoid(z1), wh1[...],
                      preferred_element_type=jnp.float32)
    a0[...] = jnp.dot(at0[...], wl0[...],
                      preferred_element_type=jnp.float32) + be0[...]
    a1[...] = jnp.dot(at1[...], wl1[...],
                      preferred_element_type=jnp.float32) + be1[...]


def _edge_mlp(gs, gd, attr0, attr1, b1r, w2t, b2r, wh0, wh1, wl0, wl1,
              be0, be1):
    be = 1000
    nb = _E // be
    rep = lambda shape: pl.BlockSpec(shape, lambda i: (0, 0))
    return pl.pallas_call(
        _edge_mlp_body,
        grid=(nb,),
        in_specs=[
            pl.BlockSpec((be, _D), lambda i: (i, 0)),
            pl.BlockSpec((be, _D), lambda i: (i, 0)),
            pl.BlockSpec((be, _D), lambda i: (i + _E // 1000, 0)),
            pl.BlockSpec((be, _D), lambda i: (i + _E // 1000, 0)),
            pl.BlockSpec((be, _EA), lambda i: (i, 0)),
            pl.BlockSpec((be, _EA), lambda i: (i, 0)),
            rep((1, _D)), rep((_D, _T)), rep((1, _T)),
            rep((_T, 1)), rep((_T, 1)), rep((_EA, 1)), rep((_EA, 1)),
            rep((1, 1)), rep((1, 1)),
        ],
        out_specs=[pl.BlockSpec((be, 1), lambda i: (i, 0))] * 4,
        out_shape=[jax.ShapeDtypeStruct((_E, 1), jnp.float32)] * 4,
    )(gs, gd, gs, gd, attr0, attr1, b1r, w2t, b2r, wh0, wh1, wl0, wl1,
      be0, be1)


def _appnp_body(sflat, br4, a4, row4, col4, maskp, alpha2,
                fill,
                x1, x2, x3, outv, tmpv, swv, misc, alob,
                sem_a, sem_b, agg, out_s):
    sba = misc.at[pl.ds(0, _NSL)]
    sbb = misc.at[pl.ds(_NSL, _NSL)]
    mbuf = misc.at[pl.ds(2 * _NSL, _CG)]
    gb0 = misc.at[pl.ds(2 * _NSL + _CG, _CG)]
    gb1 = misc.at[pl.ds(2 * _NSL + 2 * _CG, _CG)]
    c = lax.axis_index("c")
    t = lax.axis_index("s")
    pltpu.sync_copy(alpha2.at[c], alob)
    a16 = alob[0, pl.ds(0, 16)]
    om16 = alob[1, pl.ds(0, 16)]
    zero16 = jnp.zeros((16,), jnp.float32)

    def _zero_sbb():
        def zl(v, u):
            sbb[pl.ds(v * 16, 16)] = zero16
            return u
        lax.fori_loop(0, _NSL // 16, zl, 0)

    pltpu.sync_copy(br4.at[c, t], x1)
    pltpu.sync_copy(a4.at[c, t], x3)

    def _ew_chunk(k, gb):
        for v in range(_CG // 16):
            sl = pl.ds(v * 16, 16)
            val = x3[k, sl] + gb[sl]
            x3[k, sl] = 1.0 / (1.0 + jnp.exp(-val))

    pltpu.async_copy(sflat.at[x1.at[0]], gb0, sem_a)

    def gath(i, u):
        k0 = 2 * i
        pltpu.async_copy(sflat.at[x1.at[k0 + 1]], gb1, sem_b)
        pltpu.make_async_copy(sflat.at[x1.at[k0]], gb0, sem_a).wait()
        _ew_chunk(k0, gb0)

        @pl.when(k0 + 2 < _NCH)
        def _():
            pltpu.async_copy(sflat.at[x1.at[k0 + 2]], gb0, sem_a)

        pltpu.make_async_copy(sflat.at[x1.at[k0 + 1]], gb1, sem_b).wait()
        _ew_chunk(k0 + 1, gb1)
        return u
    lax.fori_loop(0, _NCH // 2, gath, 0)

    _zero_sbb()
    pltpu.sync_copy(sbb, agg.at[pl.ds(t * _NSL, _NSL)])
    plsc.subcore_barrier()
    pltpu.sync_copy(col4.at[c, t], x2)

    def scat_deg(k, u):
        pltpu.sync_copy(x3.at[k], agg.at[x2.at[k]], add=True)
        return u
    lax.fori_loop(0, _NCH, scat_deg, 0)
    plsc.subcore_barrier()

    def dblk(j, u):
        pltpu.sync_copy(agg.at[pl.ds(j * _NSL, _NSL)], sba)

        def dl(v, w):
            sl = pl.ds(v * 16, 16)
            gl = pl.ds(j * _NSL + v * 16, 16)
            d = sba[sl] + 1.0
            i = jnp.int32(0x5F3759DF) - lax.shift_right_logical(
                plsc.bitcast(d, jnp.int32), 1)
            y = plsc.bitcast(i, jnp.float32)
            y = y * (1.5 - 0.5 * d * y * y)
            y = y * (1.5 - 0.5 * d * y * y)
            y = y * (1.5 - 0.5 * d * y * y)
            tmpv[gl] = y
            swv[gl] = om16 * y * y
            return w
        lax.fori_loop(0, _NSL // 16, dl, 0)
        return u
    lax.fori_loop(0, _NS, dblk, 0)
    plsc.subcore_barrier()

    pltpu.sync_copy(row4.at[c, t], x1)

    def nl(k, u):
        for v in range(_CG // 16):
            sl = pl.ds(v * 16, 16)
            r = x1[k, sl]
            cc = x2[k, sl]
            dr = plsc.load_gather(tmpv, [r])
            dc = plsc.load_gather(tmpv, [cc])
            x3[k, sl] = om16 * dr * x3[k, sl] * dc
        return u
    lax.fori_loop(0, _NCH, nl, 0)

    pltpu.sync_copy(maskp, outv)

    def hl(v, u):
        sl = pl.ds(v * 16, 16)
        r = jnp.maximum(outv[sl], 0.0)
        outv[sl] = r
        tmpv[sl] = a16 * r
        return u
    lax.fori_loop(0, _NPAD // 16, hl, 0)

    for _hop in range(_K):
        _zero_sbb()
        pltpu.sync_copy(sbb, agg.at[pl.ds(t * _NSL, _NSL)])
        plsc.subcore_barrier()

        def scat(k, u):
            for v in range(_CG // 16):
                sl = pl.ds(v * 16, 16)
                r = x1[k, sl]
                mbuf[sl] = x3[k, sl] * plsc.load_gather(outv, [r])
            pltpu.sync_copy(mbuf, agg.at[x2.at[k]], add=True)
            return u
        lax.fori_loop(0, _NCH, scat, 0)
        plsc.subcore_barrier()

        base = t * _NSL
        pltpu.sync_copy(agg.at[pl.ds(base, _NSL)], sba)

        def upd(v, u):
            sl = pl.ds(v * 16, 16)
            gl = pl.ds(base + v * 16, 16)
            sbb[sl] = sba[sl] + swv[gl] * outv[gl] + tmpv[gl]
            return u
        lax.fori_loop(0, _NSL // 16, upd, 0)
        pltpu.sync_copy(sbb, out_s.at[pl.ds(base, _NSL)])
        plsc.subcore_barrier()
        pltpu.sync_copy(out_s, outv)

    pltpu.sync_copy(outv.at[pl.ds(t * _NSL, _NSL)],
                    fill.at[c, pl.ds(t * _NSL, _NSL)])


def _appnp(sflat, br4, a4, row4, col4, maskp, alpha2):
    mesh = plsc.VectorSubcoreMesh(
        core_axis_name="c", subcore_axis_name="s",
        num_cores=_NC, num_subcores=_NS)
    f = pl.kernel(
        _appnp_body,
        out_type=jax.ShapeDtypeStruct((_NC, _NPAD), jnp.float32),
        mesh=mesh,
        compiler_params=pltpu.CompilerParams(needs_layout_passes=False),
        scratch_types=[
            pltpu.VMEM((_NCH, _CG), jnp.int32),
            pltpu.VMEM((_NCH, _CG), jnp.int32),
            pltpu.VMEM((_NCH, _CG), jnp.float32),
            pltpu.VMEM((_NPAD,), jnp.float32),
            pltpu.VMEM((_NPAD,), jnp.float32),
            pltpu.VMEM((_NPAD,), jnp.float32),
            pltpu.VMEM((2048,), jnp.float32),
            pltpu.VMEM((2, 128), jnp.float32),
            pltpu.SemaphoreType.DMA,
            pltpu.SemaphoreType.DMA,
            pltpu.VMEM_SHARED((_NPAD,), jnp.float32),
            pltpu.VMEM_SHARED((_NPAD,), jnp.float32),
        ],
    )
    return f(sflat, br4, a4, row4, col4, maskp, alpha2)


def _pad_edges(arr, total, fill_val):
    return jnp.pad(arr, (0, total - arr.shape[0]), constant_values=fill_val)


def kernel(x, mask, dom_spatial_edge_index, dom_dom_edge_index,
           proj_spatial_edge_index, proj_dom_edge_index,
           proj_br_spatial_edge_index, proj_br_dom_edge_index,
           proj_spatial_edge_attr, proj_dom_edge_attr,
           W1, b1, W2, b2,
           pos_We, pos_be, pos_alpha, pos_bias,
           dom_We, dom_be, dom_alpha, dom_bias):
    f32, i32 = jnp.float32, jnp.int32

    w1a = W1[:, :_D]
    w1b = W1[:, _D:2 * _D]
    w1c = W1[:, 2 * _D:]
    a_nd, b_nd = _node_transform(x, (w1a + w1c).T, (w1b - w1c).T)

    src = _pad_edges(jnp.concatenate(
        [dom_spatial_edge_index[0], dom_dom_edge_index[0]]).astype(i32),
        _E2P, 0).reshape(_NW, _NCHG, _CG)
    dst = _pad_edges(jnp.concatenate(
        [dom_spatial_edge_index[1], dom_dom_edge_index[1]]).astype(i32),
        _E2P, 0).reshape(_NW, _NCHG, _CG)
    gs, gd = _gather_rows(src, dst, a_nd, b_nd)

    s_sp, s_dom, a_sp, a_dom = _edge_mlp(
        gs, gd, proj_spatial_edge_attr, proj_dom_edge_attr,
        b1.reshape(1, _D), W2.T, b2.reshape(1, _T),
        pos_We[0, _EA:].reshape(_T, 1), dom_We[0, _EA:].reshape(_T, 1),
        pos_We[0, :_EA].reshape(_EA, 1), dom_We[0, :_EA].reshape(_EA, 1),
        pos_be.reshape(1, 1), dom_be.reshape(1, 1))

    sflat = jnp.concatenate([s_sp[:, 0], s_dom[:, 0]])
    dummy = _NPAD - 1
    br4 = jnp.stack([
        _pad_edges(proj_br_spatial_edge_index.astype(i32), _EP, 0),
        _pad_edges(proj_br_dom_edge_index.astype(i32) + _E, _EP, 0),
    ]).reshape(_NC, _NS, _NCH, _CG)
    a4 = jnp.stack([
        _pad_edges(a_sp[:, 0], _EP, 0.0),
        _pad_edges(a_dom[:, 0], _EP, 0.0),
    ]).reshape(_NC, _NS, _NCH, _CG)
    row4 = jnp.stack([
        _pad_edges(proj_spatial_edge_index[0].astype(i32), _EP, dummy),
        _pad_edges(proj_dom_edge_index[0].astype(i32), _EP, dummy),
    ]).reshape(_NC, _NS, _NCH, _CG)
    col4 = jnp.stack([
        _pad_edges(proj_spatial_edge_index[1].astype(i32), _EP, dummy),
        _pad_edges(proj_dom_edge_index[1].astype(i32), _EP, dummy),
    ]).reshape(_NC, _NS, _NCH, _CG)
    maskp = jnp.pad(mask[:, 0].astype(f32), (0, _NPAD - _N))
    alph = jnp.stack([pos_alpha, dom_alpha]).astype(f32)[:, None, None]
    one128 = jnp.ones((1, 1, 128), f32)
    alob = jnp.concatenate([alph * one128, (1.0 - alph) * one128], axis=1)
    fill = _appnp(sflat, br4, a4, row4, col4, maskp, alob)

    pos_fill = fill[0, :_N, None] - jax.nn.softplus(pos_bias)
    dom_fill = fill[1, :_N, None] - jax.nn.softplus(dom_bias)
    return jnp.maximum(mask, jnp.maximum(jnp.tanh(pos_fill),
                                         jnp.tanh(dom_fill)))
